# Initial kernel scaffold; baseline (speedup 1.0000x reference)
import jax
import jax.numpy as jnp
from jax.experimental import pallas as pl


def kernel(x, edge_index, batch, W1, b1, W2, b2, bn_gamma, bn_beta, Wp, bp):
    raise NotImplementedError("write your pallas kernel here")


# speedup vs baseline: 6.0873x; 6.0873x over previous
import functools

import jax
import jax.numpy as jnp
from jax import lax
from jax.experimental import pallas as pl
from jax.experimental.pallas import tpu as pltpu
from jax.experimental.pallas import tpu_sc as plsc

N = 10000
E = 640000
D = 128
NUM_GRAPHS = 64

NC = 2
NS = 16
NW = NC * NS
CHUNK = 128
EPW = 20096
E_PAD = NW * EPW
N_ITERS = EPW // CHUNK
ROWS_PER_TILE = 632
N_PAD = NS * ROWS_PER_TILE
ACC_ROWS = N_PAD + 8


_sc_mesh = plsc.VectorSubcoreMesh(core_axis_name="c", subcore_axis_name="s")


@functools.partial(
    pl.kernel,
    out_type=jax.ShapeDtypeStruct((NC, N_PAD, D), jnp.float32),
    mesh=_sc_mesh,
    scratch_types=[
        pltpu.VMEM((CHUNK,), jnp.int32),
        pltpu.VMEM((CHUNK,), jnp.int32),
        pltpu.VMEM((CHUNK, D), jnp.float32),
        pltpu.VMEM_SHARED((ACC_ROWS, D), jnp.float32),
        pltpu.SemaphoreType.DMA,
    ],
)
def _sc_agg(h_hbm, src_hbm, dst_hbm, zeros_hbm, out_hbm, sidx, didx, rows, acc, sem):
    c = lax.axis_index("c")
    s = lax.axis_index("s")
    wid = c * NS + s
    pltpu.sync_copy(zeros_hbm, acc.at[pl.ds(s * ROWS_PER_TILE, ROWS_PER_TILE)])
    plsc.subcore_barrier()
    base = wid * EPW

    def body(i, carry):
        off = base + i * CHUNK
        pltpu.sync_copy(src_hbm.at[pl.ds(off, CHUNK)], sidx)
        pltpu.sync_copy(dst_hbm.at[pl.ds(off, CHUNK)], didx)
        pltpu.async_copy(h_hbm.at[sidx], rows, sem).wait()
        pltpu.sync_copy(rows, acc.at[didx], add=True)
        return carry

    lax.fori_loop(0, N_ITERS, body, 0)
    plsc.subcore_barrier()
    pltpu.sync_copy(
        acc.at[pl.ds(s * ROWS_PER_TILE, ROWS_PER_TILE)],
        out_hbm.at[c, pl.ds(s * ROWS_PER_TILE, ROWS_PER_TILE)],
    )


BLK = 1000
GRID = N // BLK


def _mlp_body(h_ref, a0_ref, a1_ref, w1_ref, b1_ref, w2_ref, b2_ref,
              z_ref, sum_ref, sq_ref):
    zin = h_ref[...] + a0_ref[...] + a1_ref[...]
    z1 = jnp.maximum(
        jnp.dot(zin, w1_ref[...], preferred_element_type=jnp.float32)
        + b1_ref[0:1, :], 0.0)
    z2 = jnp.maximum(
        jnp.dot(z1, w2_ref[...], preferred_element_type=jnp.float32)
        + b2_ref[0:1, :], 0.0)
    z_ref[...] = z2

    @pl.when(pl.program_id(0) == 0)
    def _():
        sum_ref[...] = jnp.zeros_like(sum_ref)
        sq_ref[...] = jnp.zeros_like(sq_ref)

    sum_ref[...] += jnp.broadcast_to(jnp.sum(z2, axis=0, keepdims=True), (8, D))
    sq_ref[...] += jnp.broadcast_to(jnp.sum(z2 * z2, axis=0, keepdims=True), (8, D))


def _mlp(h, a0, a1, w1, b1, w2, b2):
    full = pl.BlockSpec((8, D), lambda i: (0, 0))
    wfull = pl.BlockSpec((D, D), lambda i: (0, 0))
    blk = pl.BlockSpec((BLK, D), lambda i: (i, 0))
    return pl.pallas_call(
        _mlp_body,
        grid=(GRID,),
        in_specs=[blk, blk, blk, wfull, full, wfull, full],
        out_specs=[blk, full, full],
        out_shape=[
            jax.ShapeDtypeStruct((N, D), jnp.float32),
            jax.ShapeDtypeStruct((8, D), jnp.float32),
            jax.ShapeDtypeStruct((8, D), jnp.float32),
        ],
    )(h, a0, a1, w1, b1, w2, b2)


def _norm_body(first, z_ref, sum_ref, sq_ref, g_ref, be_ref, hprev_ref, out_ref):
    mu = sum_ref[0:1, :] * (1.0 / N)
    var = sq_ref[0:1, :] * (1.0 / N) - mu * mu
    inv = lax.rsqrt(var + 1e-5)
    bn = (z_ref[...] - mu) * (inv * g_ref[0:1, :]) + be_ref[0:1, :]
    if first:
        out_ref[...] = bn
    else:
        out_ref[...] = hprev_ref[...] + bn


def _norm(z, ssum, ssq, gamma, beta, hprev, first):
    full = pl.BlockSpec((8, D), lambda i: (0, 0))
    blk = pl.BlockSpec((BLK, D), lambda i: (i, 0))
    return pl.pallas_call(
        functools.partial(_norm_body, first),
        grid=(GRID,),
        in_specs=[blk, full, full, full, full, blk],
        out_specs=blk,
        out_shape=jax.ShapeDtypeStruct((N, D), jnp.float32),
    )(z, ssum, ssq, gamma, beta, hprev)


def _pool_body(h_ref, b_ref, sum_ref, cnt_ref, max_ref):
    @pl.when(pl.program_id(0) == 0)
    def _():
        sum_ref[...] = jnp.zeros_like(sum_ref)
        cnt_ref[...] = jnp.zeros_like(cnt_ref)
        max_ref[...] = jnp.full_like(max_ref, -jnp.inf)

    hb = h_ref[...]
    bid = b_ref[0, 0, :]
    segs = lax.broadcasted_iota(jnp.int32, (BLK, NUM_GRAPHS), 1)
    onehot = (bid[:, None] == segs).astype(jnp.float32)
    sum_ref[...] += jnp.dot(onehot.T, hb, preferred_element_type=jnp.float32)
    cnt_ref[...] += jnp.dot(onehot.T, jnp.ones_like(hb),
                            preferred_element_type=jnp.float32)
    upd = []
    for sgi in range(NUM_GRAPHS):
        m = jnp.where(bid[:, None] == sgi, hb, -jnp.inf)
        upd.append(jnp.max(m, axis=0))
    max_ref[...] = jnp.maximum(max_ref[...], jnp.stack(upd, axis=0))


def _pool(h, batch):
    b3 = batch.reshape(GRID, 1, BLK)
    return pl.pallas_call(
        _pool_body,
        grid=(GRID,),
        in_specs=[
            pl.BlockSpec((BLK, D), lambda i: (i, 0)),
            pl.BlockSpec((1, 1, BLK), lambda i: (i, 0, 0)),
        ],
        out_specs=[
            pl.BlockSpec((NUM_GRAPHS, D), lambda i: (0, 0)),
            pl.BlockSpec((NUM_GRAPHS, D), lambda i: (0, 0)),
            pl.BlockSpec((NUM_GRAPHS, D), lambda i: (0, 0)),
        ],
        out_shape=[
            jax.ShapeDtypeStruct((NUM_GRAPHS, D), jnp.float32),
            jax.ShapeDtypeStruct((NUM_GRAPHS, D), jnp.float32),
            jax.ShapeDtypeStruct((NUM_GRAPHS, D), jnp.float32),
        ],
    )(h, b3)


def _final_body(sum_ref, cnt_ref, max_ref, wp_ref, bp_ref, out_ref):
    mean = sum_ref[...] / jnp.maximum(cnt_ref[...], 1.0)
    wp = wp_ref[...]
    out_ref[...] = (
        jnp.dot(mean, wp[:D, :], preferred_element_type=jnp.float32)
        + jnp.dot(max_ref[...], wp[D:, :], preferred_element_type=jnp.float32)
        + bp_ref[0:1, :]
    )


def _final(ssum, cnt, smax, wp, bp):
    return pl.pallas_call(
        _final_body,
        out_shape=jax.ShapeDtypeStruct((NUM_GRAPHS, D), jnp.float32),
    )(ssum, cnt, smax, wp, bp)


def kernel(x, edge_index, batch, W1, b1, W2, b2, bn_gamma, bn_beta, Wp, bp):
    src = edge_index[0].astype(jnp.int32)
    dst = edge_index[1].astype(jnp.int32)
    pad = E_PAD - E
    src_p = jnp.concatenate([src, jnp.zeros((pad,), jnp.int32)])
    dst_p = jnp.concatenate([dst, jnp.full((pad,), N_PAD, jnp.int32)])
    zeros = jnp.zeros((ROWS_PER_TILE, D), jnp.float32)

    def row8(v):
        return jnp.broadcast_to(v.reshape(1, D), (8, D))

    h = x
    for i in range(5):
        agg = _sc_agg(h, src_p, dst_p, zeros)
        z, ssum, ssq = _mlp(h, agg[0, :N], agg[1, :N], W1[i], row8(b1[i]), W2[i],
                            row8(b2[i]))
        h = _norm(z, ssum, ssq, row8(bn_gamma[i]), row8(bn_beta[i]), h, first=(i == 0))

    ssum, cnt, smax = _pool(h, batch.astype(jnp.int32))
    bp8 = jnp.broadcast_to(bp.reshape(1, D), (8, D))
    return _final(ssum, cnt, smax, Wp, bp8)


# speedup vs baseline: 6.1300x; 1.0070x over previous
import functools

import jax
import jax.numpy as jnp
from jax import lax
from jax.experimental import pallas as pl
from jax.experimental.pallas import tpu as pltpu
from jax.experimental.pallas import tpu_sc as plsc

N = 10000
E = 640000
D = 128
NUM_GRAPHS = 64

NC = 2
NS = 16
NW = NC * NS
CHUNK = 128
EPW = 20224
E_PAD = NW * EPW
E_ALLOC = E_PAD + CHUNK
N_ITERS = EPW // CHUNK
ROWS_PER_TILE = 632
N_PAD = NS * ROWS_PER_TILE
ACC_ROWS = N_PAD + 8


_sc_mesh = plsc.VectorSubcoreMesh(core_axis_name="c", subcore_axis_name="s")


@functools.partial(
    pl.kernel,
    out_type=jax.ShapeDtypeStruct((NC, N_PAD, D), jnp.float32),
    mesh=_sc_mesh,
    scratch_types=[
        pltpu.VMEM((CHUNK,), jnp.int32),
        pltpu.VMEM((CHUNK,), jnp.int32),
        pltpu.VMEM((CHUNK,), jnp.int32),
        pltpu.VMEM((CHUNK,), jnp.int32),
        pltpu.VMEM((CHUNK, D), jnp.float32),
        pltpu.VMEM((CHUNK, D), jnp.float32),
        pltpu.VMEM_SHARED((ACC_ROWS, D), jnp.float32),
        pltpu.SemaphoreType.DMA,
        pltpu.SemaphoreType.DMA,
    ],
)
def _sc_agg(h_hbm, src_hbm, dst_hbm, zeros_hbm, out_hbm,
            sidx0, sidx1, didx0, didx1, rows0, rows1, acc, sem0, sem1):
    c = lax.axis_index("c")
    s = lax.axis_index("s")
    wid = c * NS + s
    pltpu.sync_copy(zeros_hbm, acc.at[pl.ds(s * ROWS_PER_TILE, ROWS_PER_TILE)])
    plsc.subcore_barrier()
    base = wid * EPW
    sidx = (sidx0, sidx1)
    didx = (didx0, didx1)
    rows = (rows0, rows1)
    sems = (sem0, sem1)

    pltpu.sync_copy(src_hbm.at[pl.ds(base, CHUNK)], sidx0)
    pltpu.async_copy(h_hbm.at[sidx0], rows0, sem0)

    def body(i2, carry):
        for p in range(2):
            q = 1 - p
            off = base + i2 * (2 * CHUNK) + p * CHUNK
            pltpu.sync_copy(src_hbm.at[pl.ds(off + CHUNK, CHUNK)], sidx[q])
            pltpu.async_copy(h_hbm.at[sidx[q]], rows[q], sems[q])
            pltpu.sync_copy(dst_hbm.at[pl.ds(off, CHUNK)], didx[p])
            pltpu.make_async_copy(h_hbm.at[sidx[p]], rows[p], sems[p]).wait()
            pltpu.sync_copy(rows[p], acc.at[didx[p]], add=True)
        return carry

    lax.fori_loop(0, N_ITERS // 2, body, 0)
    pltpu.make_async_copy(h_hbm.at[sidx0], rows0, sem0).wait()
    plsc.subcore_barrier()
    pltpu.sync_copy(
        acc.at[pl.ds(s * ROWS_PER_TILE, ROWS_PER_TILE)],
        out_hbm.at[c, pl.ds(s * ROWS_PER_TILE, ROWS_PER_TILE)],
    )


BLK = 1000
GRID = N // BLK


def _mlp_body(h_ref, a0_ref, a1_ref, w1_ref, b1_ref, w2_ref, b2_ref,
              z_ref, sum_ref, sq_ref):
    zin = h_ref[...] + a0_ref[...] + a1_ref[...]
    z1 = jnp.maximum(
        jnp.dot(zin, w1_ref[...], preferred_element_type=jnp.float32)
        + b1_ref[0:1, :], 0.0)
    z2 = jnp.maximum(
        jnp.dot(z1, w2_ref[...], preferred_element_type=jnp.float32)
        + b2_ref[0:1, :], 0.0)
    z_ref[...] = z2

    @pl.when(pl.program_id(0) == 0)
    def _():
        sum_ref[...] = jnp.zeros_like(sum_ref)
        sq_ref[...] = jnp.zeros_like(sq_ref)

    sum_ref[...] += jnp.broadcast_to(jnp.sum(z2, axis=0, keepdims=True), (8, D))
    sq_ref[...] += jnp.broadcast_to(jnp.sum(z2 * z2, axis=0, keepdims=True), (8, D))


def _mlp(h, a0, a1, w1, b1, w2, b2):
    full = pl.BlockSpec((8, D), lambda i: (0, 0))
    wfull = pl.BlockSpec((D, D), lambda i: (0, 0))
    blk = pl.BlockSpec((BLK, D), lambda i: (i, 0))
    return pl.pallas_call(
        _mlp_body,
        grid=(GRID,),
        in_specs=[blk, blk, blk, wfull, full, wfull, full],
        out_specs=[blk, full, full],
        out_shape=[
            jax.ShapeDtypeStruct((N, D), jnp.float32),
            jax.ShapeDtypeStruct((8, D), jnp.float32),
            jax.ShapeDtypeStruct((8, D), jnp.float32),
        ],
    )(h, a0, a1, w1, b1, w2, b2)


def _norm_body(first, z_ref, sum_ref, sq_ref, g_ref, be_ref, hprev_ref, out_ref):
    mu = sum_ref[0:1, :] * (1.0 / N)
    var = sq_ref[0:1, :] * (1.0 / N) - mu * mu
    inv = lax.rsqrt(var + 1e-5)
    bn = (z_ref[...] - mu) * (inv * g_ref[0:1, :]) + be_ref[0:1, :]
    if first:
        out_ref[...] = bn
    else:
        out_ref[...] = hprev_ref[...] + bn


def _norm(z, ssum, ssq, gamma, beta, hprev, first):
    full = pl.BlockSpec((8, D), lambda i: (0, 0))
    blk = pl.BlockSpec((BLK, D), lambda i: (i, 0))
    return pl.pallas_call(
        functools.partial(_norm_body, first),
        grid=(GRID,),
        in_specs=[blk, full, full, full, full, blk],
        out_specs=blk,
        out_shape=jax.ShapeDtypeStruct((N, D), jnp.float32),
    )(z, ssum, ssq, gamma, beta, hprev)


def _pool_body(h_ref, b_ref, sum_ref, cnt_ref, max_ref):
    @pl.when(pl.program_id(0) == 0)
    def _():
        sum_ref[...] = jnp.zeros_like(sum_ref)
        cnt_ref[...] = jnp.zeros_like(cnt_ref)
        max_ref[...] = jnp.full_like(max_ref, -jnp.inf)

    hb = h_ref[...]
    bid = b_ref[0, 0, :]
    segs = lax.broadcasted_iota(jnp.int32, (BLK, NUM_GRAPHS), 1)
    onehot = (bid[:, None] == segs).astype(jnp.float32)
    sum_ref[...] += jnp.dot(onehot.T, hb, preferred_element_type=jnp.float32)
    cnt_ref[...] += jnp.dot(onehot.T, jnp.ones_like(hb),
                            preferred_element_type=jnp.float32)
    upd = []
    for sgi in range(NUM_GRAPHS):
        m = jnp.where(bid[:, None] == sgi, hb, -jnp.inf)
        upd.append(jnp.max(m, axis=0))
    max_ref[...] = jnp.maximum(max_ref[...], jnp.stack(upd, axis=0))


def _pool(h, batch):
    b3 = batch.reshape(GRID, 1, BLK)
    return pl.pallas_call(
        _pool_body,
        grid=(GRID,),
        in_specs=[
            pl.BlockSpec((BLK, D), lambda i: (i, 0)),
            pl.BlockSpec((1, 1, BLK), lambda i: (i, 0, 0)),
        ],
        out_specs=[
            pl.BlockSpec((NUM_GRAPHS, D), lambda i: (0, 0)),
            pl.BlockSpec((NUM_GRAPHS, D), lambda i: (0, 0)),
            pl.BlockSpec((NUM_GRAPHS, D), lambda i: (0, 0)),
        ],
        out_shape=[
            jax.ShapeDtypeStruct((NUM_GRAPHS, D), jnp.float32),
            jax.ShapeDtypeStruct((NUM_GRAPHS, D), jnp.float32),
            jax.ShapeDtypeStruct((NUM_GRAPHS, D), jnp.float32),
        ],
    )(h, b3)


def _final_body(sum_ref, cnt_ref, max_ref, wp_ref, bp_ref, out_ref):
    mean = sum_ref[...] / jnp.maximum(cnt_ref[...], 1.0)
    wp = wp_ref[...]
    out_ref[...] = (
        jnp.dot(mean, wp[:D, :], preferred_element_type=jnp.float32)
        + jnp.dot(max_ref[...], wp[D:, :], preferred_element_type=jnp.float32)
        + bp_ref[0:1, :]
    )


def _final(ssum, cnt, smax, wp, bp):
    return pl.pallas_call(
        _final_body,
        out_shape=jax.ShapeDtypeStruct((NUM_GRAPHS, D), jnp.float32),
    )(ssum, cnt, smax, wp, bp)


def kernel(x, edge_index, batch, W1, b1, W2, b2, bn_gamma, bn_beta, Wp, bp):
    src = edge_index[0].astype(jnp.int32)
    dst = edge_index[1].astype(jnp.int32)
    pad = E_ALLOC - E
    src_p = jnp.concatenate([src, jnp.zeros((pad,), jnp.int32)])
    dst_p = jnp.concatenate([dst, jnp.full((pad,), N_PAD, jnp.int32)])
    zeros = jnp.zeros((ROWS_PER_TILE, D), jnp.float32)

    def row8(v):
        return jnp.broadcast_to(v.reshape(1, D), (8, D))

    h = x
    for i in range(5):
        agg = _sc_agg(h, src_p, dst_p, zeros)
        z, ssum, ssq = _mlp(h, agg[0, :N], agg[1, :N], W1[i], row8(b1[i]), W2[i],
                            row8(b2[i]))
        h = _norm(z, ssum, ssq, row8(bn_gamma[i]), row8(bn_beta[i]), h, first=(i == 0))

    ssum, cnt, smax = _pool(h, batch.astype(jnp.int32))
    bp8 = jnp.broadcast_to(bp.reshape(1, D), (8, D))
    return _final(ssum, cnt, smax, Wp, bp8)
